# baseline (speedup 1.0000x reference)
import jax
import jax.numpy as jnp
from jax.experimental import pallas as pl
from jax.experimental.pallas import tpu as pltpu

_N0, _N1, _D0, _D1, _H = 10000, 5000, 128, 128, 64
_TILE_I = 400


def _fused_body(fea0_ref, fea1_ref, adj_ref, W0_ref, W1_ref, Wp_ref,
                b01_ref, bp_ref, logits_ref, out0_ref, out1_ref, p1_scr):
    i = pl.program_id(0)

    @pl.when(i == 0)
    def _init():
        p1 = jnp.dot(fea1_ref[...], W1_ref[...],
                     preferred_element_type=jnp.float32)
        p1_scr[...] = p1
        out1_ref[...] = p1 + b01_ref[...]

    adj = adj_ref[...]
    p0 = jnp.dot(fea0_ref[...], W0_ref[...],
                 preferred_element_type=jnp.float32)
    o0 = (jnp.dot(adj, p1_scr[...], preferred_element_type=jnp.float32)
          + p0 + b01_ref[...])
    out0_ref[...] = o0
    out1_ref[...] += jax.lax.dot_general(
        adj, p0, (((0,), (0,)), ((), ())),
        preferred_element_type=jnp.float32)
    z = jnp.maximum(o0, 0.0)
    logits_ref[...] = (jnp.dot(z, Wp_ref[...],
                               preferred_element_type=jnp.float32)
                       + bp_ref[...])


def kernel(fea_0, fea_1, adj_01, adj_masks, W0, b0, W1, b1, Wp, bp):
    del adj_masks
    b01 = (b0 + b1).reshape(1, _H)
    bp2 = bp.reshape(1, _D1)
    grid = (_N0 // _TILE_I,)

    logits, out0, out1 = pl.pallas_call(
        _fused_body,
        grid=grid,
        in_specs=[
            pl.BlockSpec((_TILE_I, _D0), lambda i: (i, 0)),
            pl.BlockSpec((_N1, _D1), lambda i: (0, 0)),
            pl.BlockSpec((_TILE_I, _N1), lambda i: (i, 0)),
            pl.BlockSpec((_D0, _H), lambda i: (0, 0)),
            pl.BlockSpec((_D1, _H), lambda i: (0, 0)),
            pl.BlockSpec((_H, _D1), lambda i: (0, 0)),
            pl.BlockSpec((1, _H), lambda i: (0, 0)),
            pl.BlockSpec((1, _D1), lambda i: (0, 0)),
        ],
        out_specs=[
            pl.BlockSpec((_TILE_I, _D1), lambda i: (i, 0)),
            pl.BlockSpec((_TILE_I, _H), lambda i: (i, 0)),
            pl.BlockSpec((_N1, _H), lambda i: (0, 0)),
        ],
        out_shape=[
            jax.ShapeDtypeStruct((_N0, _D1), jnp.float32),
            jax.ShapeDtypeStruct((_N0, _H), jnp.float32),
            jax.ShapeDtypeStruct((_N1, _H), jnp.float32),
        ],
        scratch_shapes=[pltpu.VMEM((_N1, _H), jnp.float32)],
        compiler_params=pltpu.CompilerParams(
            dimension_semantics=("arbitrary",)),
    )(fea_0, fea_1, adj_01, W0, W1, Wp, b01, bp2)

    return logits, out0, out1
